# baseline (speedup 1.0000x reference)
import functools

import jax
import jax.numpy as jnp
from jax import lax
from jax.experimental import pallas as pl
from jax.experimental.pallas import tpu as pltpu
from jax.experimental.pallas import tpu_sc as plsc

B, F, V, E = 4096, 26, 1000000, 16
BF = B * F
FE = F * E
H = 400
EPS = 1e-3

NC, NS = 2, 16
NW = NC * NS
B_PER_W = BF // NW
CHUNK = 128
NCH = B_PER_W // CHUNK
PAD_NCH = 32
G = 128
VG = V // 8
FWG = (V + G - 1) // G
RBLK = 2048
BLK = 512
assert B_PER_W % CHUNK == 0 and BF % RBLK == 0 and B % BLK == 0


def _sc_gather(idx_hi8, idx_hi128, emb8, fw128):
    mesh = plsc.VectorSubcoreMesh(core_axis_name="c", subcore_axis_name="s")

    @functools.partial(
        pl.kernel,
        out_type=[
            jax.ShapeDtypeStruct((BF, G), jnp.float32),
            jax.ShapeDtypeStruct((BF, G), jnp.float32),
        ],
        mesh=mesh,
        scratch_types=[
            pltpu.VMEM((PAD_NCH, CHUNK), jnp.int32),
            pltpu.VMEM((PAD_NCH, CHUNK), jnp.int32),
            pltpu.VMEM((CHUNK, G), jnp.float32),
            pltpu.VMEM((CHUNK, G), jnp.float32),
            pltpu.SemaphoreType.DMA,
            pltpu.SemaphoreType.DMA,
        ],
    )
    def sc_kernel(i8_hbm, i128_hbm, emb_hbm, fw_hbm, ge_hbm, fwg_hbm,
                  i8_v, i128_v, buf_e, buf_f, sem1, sem2):
        wid = lax.axis_index("s") * NC + lax.axis_index("c")
        row0 = wid * PAD_NCH
        pltpu.sync_copy(i8_hbm.at[pl.ds(row0, PAD_NCH)], i8_v)
        pltpu.sync_copy(i128_hbm.at[pl.ds(row0, PAD_NCH)], i128_v)
        base = wid * B_PER_W

        @pl.loop(0, NCH)
        def _(j):
            c1 = pltpu.async_copy(emb_hbm.at[i8_v.at[j]], buf_e, sem1)
            c2 = pltpu.async_copy(fw_hbm.at[i128_v.at[j]], buf_f, sem2)
            c1.wait()
            c2.wait()
            out = pl.ds(base + j * CHUNK, CHUNK)
            pltpu.sync_copy(buf_e, ge_hbm.at[out])
            pltpu.sync_copy(buf_f, fwg_hbm.at[out])

    return sc_kernel(idx_hi8, idx_hi128, emb8, fw128)


def _select_body(ge_ref, fwg_ref, lo8_ref, pos_ref, fv_ref, c8_ref,
                 fev_ref, yfw_ref):
    f32 = jnp.float32
    hi = lax.Precision.HIGHEST
    lane = lax.broadcasted_iota(jnp.int32, (RBLK, G), 1)
    slot = (lane // E).astype(f32)
    masked = ge_ref[...] * (slot == lo8_ref[...]).astype(f32)
    fev = lax.dot_general(masked, c8_ref[...], (((1,), (0,)), ((), ())),
                          precision=hi, preferred_element_type=f32)
    fev_ref[...] = fev * fv_ref[...]
    fsel = fwg_ref[...] * (lane.astype(f32) == pos_ref[...]).astype(f32)
    yfw_ref[...] = jnp.sum(fsel, axis=1, keepdims=True) * fv_ref[...]


def _mlp_body(fev_ref, yfw_ref, d1_ref, b1_ref, d2_ref, b2_ref,
              d3_ref, b3_ref, s_ref, w1_ref, w2_ref, w3_ref, bias_ref,
              out_ref):
    f32 = jnp.float32
    hi = lax.Precision.HIGHEST
    fev = fev_ref[...]

    acc = lax.dot_general(yfw_ref[...], w1_ref[...], (((1,), (0,)), ((), ())),
                          precision=hi, preferred_element_type=f32)

    summed = lax.dot_general(fev, s_ref[...], (((1,), (0,)), ((), ())),
                             precision=hi, preferred_element_type=f32)
    part2 = lax.dot_general(fev * fev, s_ref[...], (((1,), (0,)), ((), ())),
                            precision=hi, preferred_element_type=f32)
    y2 = 0.5 * (summed * summed - part2)
    acc += lax.dot_general(y2, w2_ref[...], (((1,), (0,)), ((), ())),
                           precision=hi, preferred_element_type=f32)

    h = lax.dot_general(fev, d1_ref[...], (((1,), (0,)), ((), ())),
                        precision=hi, preferred_element_type=f32)
    h = jnp.maximum(h + b1_ref[...], 0.0)
    h = lax.dot_general(h, d2_ref[...], (((1,), (0,)), ((), ())),
                        precision=hi, preferred_element_type=f32)
    h = jnp.maximum(h + b2_ref[...], 0.0)
    h = lax.dot_general(h, d3_ref[...], (((1,), (0,)), ((), ())),
                        precision=hi, preferred_element_type=f32)
    h = jnp.maximum(h + b3_ref[...], 0.0)
    acc += lax.dot_general(h, w3_ref[...], (((1,), (0,)), ((), ())),
                           precision=hi, preferred_element_type=f32)
    out_ref[...] = acc + bias_ref[...]


def kernel(feat_index, feat_value, first_w, emb, d1_k, d1_b, bn1_g, bn1_b,
           d2_k, d2_b, bn2_g, bn2_b, d3_k, d3_b, bn3_g, bn3_b, fc_k, fc_b):
    f32 = jnp.float32
    idx = feat_index.reshape(NW, NCH, CHUNK).astype(jnp.int32)
    idx = jnp.pad(idx, ((0, 0), (0, PAD_NCH - NCH), (0, 0)))
    idx = idx.reshape(NW * PAD_NCH, CHUNK)
    idx_hi8 = idx // 8
    idx_hi128 = idx // G

    emb8 = emb.reshape(VG, G)
    fw_flat = jnp.pad(first_w.reshape(-1), (0, FWG * G - V))
    fw128 = fw_flat.reshape(FWG, G)

    ge128, fwg = _sc_gather(idx_hi8, idx_hi128, emb8, fw128)

    idx_flat = feat_index.reshape(BF, 1).astype(jnp.int32)
    lo8 = (idx_flat % 8).astype(f32)
    pos = (idx_flat % G).astype(f32)
    fv_flat = feat_value.reshape(BF, 1)
    c8 = (lax.broadcasted_iota(jnp.int32, (G, E), 0) % E ==
          lax.broadcasted_iota(jnp.int32, (G, E), 1)).astype(f32)

    grid_s = (BF // RBLK,)
    rspec = lambda w: pl.BlockSpec((RBLK, w), lambda i: (i, 0))
    fev, yfw = pl.pallas_call(
        _select_body,
        grid=grid_s,
        in_specs=[
            rspec(G), rspec(G), rspec(1), rspec(1), rspec(1),
            pl.BlockSpec((G, E), lambda i: (0, 0)),
        ],
        out_specs=[rspec(E), rspec(1)],
        out_shape=[
            jax.ShapeDtypeStruct((BF, E), f32),
            jax.ShapeDtypeStruct((BF, 1), f32),
        ],
    )(ge128, fwg, lo8, pos, fv_flat, c8)

    fev = fev.reshape(B, FE)
    yfw = yfw.reshape(B, F)

    inv = 1.0 / jnp.sqrt(1.0 + EPS)
    d1 = d1_k * (bn1_g * inv)[None, :]
    b1 = (d1_b * bn1_g * inv + bn1_b)[None, :]
    d2 = d2_k * (bn2_g * inv)[None, :]
    b2 = (d2_b * bn2_g * inv + bn2_b)[None, :]
    d3 = d3_k * (bn3_g * inv)[None, :]
    b3 = (d3_b * bn3_g * inv + bn3_b)[None, :]

    w1 = fc_k[0:F, :]
    w2 = fc_k[F:F + E, :]
    w3 = fc_k[F + E:, :]
    bias = fc_b[None, :]

    s = (lax.broadcasted_iota(jnp.int32, (FE, E), 0) % E ==
         lax.broadcasted_iota(jnp.int32, (FE, E), 1)).astype(f32)

    grid = (B // BLK,)
    bspec = lambda w: pl.BlockSpec((BLK, w), lambda i: (i, 0))
    wspec = lambda shp: pl.BlockSpec(shp, lambda i: (0, 0))

    out = pl.pallas_call(
        _mlp_body,
        grid=grid,
        in_specs=[
            bspec(FE), bspec(F),
            wspec((FE, H)), wspec((1, H)),
            wspec((H, H)), wspec((1, H)),
            wspec((H, H)), wspec((1, H)),
            wspec((FE, E)),
            wspec((F, 1)), wspec((E, 1)), wspec((H, 1)), wspec((1, 1)),
        ],
        out_specs=pl.BlockSpec((BLK, 1), lambda i: (i, 0)),
        out_shape=jax.ShapeDtypeStruct((B, 1), f32),
    )(fev, yfw, d1, b1, d2, b2, d3, b3, s, w1, w2, w3, bias)
    return out
